# baseline (speedup 1.0000x reference)
import functools

import jax
import jax.numpy as jnp
from jax import lax
from jax.experimental import pallas as pl
from jax.experimental.pallas import tpu as pltpu
from jax.experimental.pallas import tpu_sc as plsc

_NUM_POS = 1500
_DIM = 1024
_TOTAL = _NUM_POS * _DIM

_NC = 2
_NS = 16
_NW = _NC * _NS
_CHUNK = _TOTAL // _NW

_mesh = plsc.VectorSubcoreMesh(core_axis_name="c", subcore_axis_name="s")


@functools.partial(
    pl.kernel,
    mesh=_mesh,
    out_type=jax.ShapeDtypeStruct((_TOTAL,), jnp.float32),
)
def _copy_kernel(w_hbm, out_hbm):
    wid = lax.axis_index("s") * _NC + lax.axis_index("c")
    base = wid * _CHUNK
    pltpu.sync_copy(
        w_hbm.at[pl.ds(base, _CHUNK)],
        out_hbm.at[pl.ds(base, _CHUNK)],
    )


def kernel(input_ids, weight):
    del input_ids
    flat = weight.reshape(_TOTAL)
    return _copy_kernel(flat).reshape(_NUM_POS, _DIM)


# speedup vs baseline: 6.0951x; 6.0951x over previous
import functools

import jax
import jax.numpy as jnp
from jax import lax
from jax.experimental import pallas as pl
from jax.experimental.pallas import tpu as pltpu
from jax.experimental.pallas import tpu_sc as plsc

_NUM_POS = 1500
_DIM = 1024
_TOTAL = _NUM_POS * _DIM

_NC = 2
_NS = 16
_NW = _NC * _NS
_CHUNK = _TOTAL // _NW

_mesh = plsc.VectorSubcoreMesh(core_axis_name="c", subcore_axis_name="s")


@functools.partial(
    pl.kernel,
    mesh=_mesh,
    out_type=jax.ShapeDtypeStruct((_TOTAL,), jnp.float32),
    scratch_types=[pltpu.VMEM((_CHUNK,), jnp.float32)],
)
def _copy_kernel(w_hbm, out_hbm, buf):
    wid = lax.axis_index("s") * _NC + lax.axis_index("c")
    base = wid * _CHUNK
    pltpu.sync_copy(w_hbm.at[pl.ds(base, _CHUNK)], buf)
    pltpu.sync_copy(buf, out_hbm.at[pl.ds(base, _CHUNK)])


def kernel(input_ids, weight):
    del input_ids
    flat = weight.reshape(_TOTAL)
    return _copy_kernel(flat).reshape(_NUM_POS, _DIM)
